# baseline (speedup 1.0000x reference)
import jax
import jax.numpy as jnp
import numpy as np
from jax import lax
from jax.experimental import pallas as pl
from jax.experimental.pallas import tpu as pltpu

N = 10000
C = 256
K = 5000
KP = 5120
NP = 10240
NEG_SLOPE = 0.2


def _mm_body(a_ref, b_ref, o_ref):
    @pl.when(pl.program_id(2) == 0)
    def _init():
        o_ref[...] = jnp.zeros_like(o_ref)
    o_ref[...] += jnp.dot(a_ref[...], b_ref[...],
                          preferred_element_type=jnp.float32)


def _mm(a, b, bm, bk, bn):
    m, kd = a.shape
    _, n = b.shape
    return pl.pallas_call(
        _mm_body,
        grid=(m // bm, n // bn, kd // bk),
        in_specs=[pl.BlockSpec((bm, bk), lambda i, j, k: (i, k)),
                  pl.BlockSpec((bk, bn), lambda i, j, k: (k, j))],
        out_specs=pl.BlockSpec((bm, bn), lambda i, j, k: (i, j)),
        out_shape=jax.ShapeDtypeStruct((m, n), jnp.float32),
        compiler_params=pltpu.CompilerParams(
            dimension_semantics=("parallel", "parallel", "arbitrary")),
    )(a, b)


def kernel(x, edge_index, Wq, bq, Wa, ba, W1, b1, W2, W3, b3):
    n = x.shape[0]
    loops = jnp.arange(n, dtype=edge_index.dtype)
    ei = jnp.concatenate([edge_index, jnp.stack([loops, loops])], axis=1)
    row, col = ei[0], ei[1]
    xpj = x[col]
    Xq = jax.ops.segment_max(xpj, row, num_segments=n)
    Mq = Xq @ Wq + bq
    Mqe = Mq[row]
    sc = jnp.concatenate([Mqe, xpj], axis=-1) @ Wa + ba
    sc = jax.nn.leaky_relu(sc, negative_slope=NEG_SLOPE)
    s1 = sc[:, 0]
    smax = jax.ops.segment_max(s1, row, num_segments=n)
    ex = jnp.exp(s1 - smax[row])
    den = jax.ops.segment_sum(ex, row, num_segments=n)
    score = ex / den[row]
    vj = x[col] * score[:, None]
    out = jax.ops.segment_sum(vj, row, num_segments=n)
    a = out @ W1 + b1
    bb = out @ W2
    msg = a[col] - bb[row]
    agg = jax.ops.segment_sum(msg, row, num_segments=n)
    fitness = jax.nn.sigmoid(agg + out @ W3 + b3)[:, 0]
    _, perm = jax.lax.top_k(fitness, K)
    x_out = out[perm] * fitness[perm][:, None]

    rank = jnp.full((n,), K, jnp.int32).at[perm].set(
        jnp.arange(K, dtype=jnp.int32))
    cle = rank[row]
    w = jnp.where(cle < K, score, 0.0)
    cle = jnp.minimum(cle, KP - 1)
    S = jnp.zeros((NP, KP), jnp.float32).at[(col, cle)].add(w)
    A = jnp.zeros((NP, NP), jnp.bfloat16).at[(row, col)].add(
        jnp.ones((row.shape[0],), jnp.bfloat16))
    Sb = S.astype(jnp.bfloat16)
    B = _mm(A, Sb, 512, 1024, 512)
    St = jnp.swapaxes(Sb, 0, 1)
    Em_full = _mm(St, B.astype(jnp.bfloat16), 512, 1024, 512)
    Em = Em_full[:K, :K]
    dix = jnp.arange(K)
    Em = Em.at[dix, dix].set(1.0)

    batch_out = jnp.zeros((K,), jnp.int32)
    return (x_out, Em, batch_out, perm)


# speedup vs baseline: 1.0631x; 1.0631x over previous
import jax
import jax.numpy as jnp
import numpy as np
from jax import lax
from jax.experimental import pallas as pl
from jax.experimental.pallas import tpu as pltpu

N = 10000
C = 256
K = 5000
KP = 5120
NP = 10240
NEG_SLOPE = 0.2


def _mm_body(a_ref, b_ref, o_ref, acc_ref):
    @pl.when(pl.program_id(2) == 0)
    def _init():
        acc_ref[...] = jnp.zeros_like(acc_ref)
    acc_ref[...] += jnp.dot(a_ref[...], b_ref[...],
                            preferred_element_type=jnp.float32)

    @pl.when(pl.program_id(2) == pl.num_programs(2) - 1)
    def _flush():
        o_ref[...] = acc_ref[...].astype(o_ref.dtype)


def _mm(a, b, bm, bk, bn, out_dtype):
    m, kd = a.shape
    _, n = b.shape
    return pl.pallas_call(
        _mm_body,
        grid=(m // bm, n // bn, kd // bk),
        in_specs=[pl.BlockSpec((bm, bk), lambda i, j, k: (i, k)),
                  pl.BlockSpec((bk, bn), lambda i, j, k: (k, j))],
        out_specs=pl.BlockSpec((bm, bn), lambda i, j, k: (i, j)),
        out_shape=jax.ShapeDtypeStruct((m, n), out_dtype),
        scratch_shapes=[pltpu.VMEM((bm, bn), jnp.float32)],
        compiler_params=pltpu.CompilerParams(
            dimension_semantics=("parallel", "parallel", "arbitrary")),
    )(a, b)


def kernel(x, edge_index, Wq, bq, Wa, ba, W1, b1, W2, W3, b3):
    n = x.shape[0]
    loops = jnp.arange(n, dtype=edge_index.dtype)
    ei = jnp.concatenate([edge_index, jnp.stack([loops, loops])], axis=1)
    row, col = ei[0], ei[1]
    xpj = x[col]
    Xq = jax.ops.segment_max(xpj, row, num_segments=n)
    Mq = Xq @ Wq + bq
    Mqe = Mq[row]
    sc = jnp.concatenate([Mqe, xpj], axis=-1) @ Wa + ba
    sc = jax.nn.leaky_relu(sc, negative_slope=NEG_SLOPE)
    s1 = sc[:, 0]
    smax = jax.ops.segment_max(s1, row, num_segments=n)
    ex = jnp.exp(s1 - smax[row])
    den = jax.ops.segment_sum(ex, row, num_segments=n)
    score = ex / den[row]
    vj = x[col] * score[:, None]
    out = jax.ops.segment_sum(vj, row, num_segments=n)
    a = out @ W1 + b1
    bb = out @ W2
    msg = a[col] - bb[row]
    agg = jax.ops.segment_sum(msg, row, num_segments=n)
    fitness = jax.nn.sigmoid(agg + out @ W3 + b3)[:, 0]
    _, perm = jax.lax.top_k(fitness, K)
    x_out = out[perm] * fitness[perm][:, None]

    rank = jnp.full((n,), K, jnp.int32).at[perm].set(
        jnp.arange(K, dtype=jnp.int32))
    cle = rank[row]
    w = jnp.where(cle < K, score, 0.0)
    cle = jnp.minimum(cle, KP - 1)
    Sb = jnp.zeros((NP, KP), jnp.bfloat16).at[(col, cle)].add(
        w.astype(jnp.bfloat16))
    A = jnp.zeros((NP, NP), jnp.bfloat16).at[(row, col)].add(
        jnp.ones((row.shape[0],), jnp.bfloat16))
    B = _mm(A, Sb, 1024, 2048, 1024, jnp.bfloat16)
    St = jnp.swapaxes(Sb, 0, 1)
    Em_full = _mm(St, B, 1024, 2048, 1024, jnp.float32)
    Em = Em_full[:K, :K]
    dix = jnp.arange(K)
    Em = Em.at[dix, dix].set(1.0)

    batch_out = jnp.zeros((K,), jnp.int32)
    return (x_out, Em, batch_out, perm)
